# Initial kernel scaffold; baseline (speedup 1.0000x reference)
import jax
import jax.numpy as jnp
from jax.experimental import pallas as pl


def kernel(feature_memory, entropy_memory, logits, image_features_global):
    raise NotImplementedError("write your pallas kernel here")


# speedup vs baseline: 2.7640x; 2.7640x over previous
import functools

import jax
import jax.numpy as jnp
from jax import lax
from jax.experimental import pallas as pl
from jax.experimental.pallas import tpu as pltpu

_C = 1000
_M = 32
_D = 1024
_B = 4096
_BBLK = 256
_NSTEPS = _B // _BBLK


def _stats_kernel(logits_ref, emt_ref, wplus_ref, slot_ref, do_ref, cnt_ref,
                  entwin_ref, worst_ref):
    i = pl.program_id(0)

    @pl.when(i == 0)
    def _init():
        wplus_ref[...] = jnp.zeros_like(wplus_ref)
        entwin_ref[...] = jnp.zeros_like(entwin_ref)
        emt = emt_ref[...]
        w = jnp.max(emt, axis=0, keepdims=True)
        sub = lax.broadcasted_iota(jnp.int32, (_M, _C), 0)
        slot_ref[...] = jnp.min(jnp.where(emt == w, sub, _M), axis=0,
                                keepdims=True)
        worst_ref[...] = w

    l = logits_ref[...]
    m = jnp.max(l, axis=1, keepdims=True)
    e = jnp.exp(l - m)
    z = jnp.sum(e, axis=1, keepdims=True)
    p = e / z
    ent = -jnp.sum(p * jnp.log(p + 1e-8), axis=1, keepdims=True)
    lane = lax.broadcasted_iota(jnp.int32, (_BBLK, _C), 1)
    pseudo = jnp.min(jnp.where(l == m, lane, _C), axis=1, keepdims=True)
    onehot = lane == pseudo
    row = lax.broadcasted_iota(jnp.int32, (_BBLK, 1), 0)
    bplus = i * _BBLK + row + 1
    wblk = jnp.max(jnp.where(onehot, bplus, 0), axis=0, keepdims=True)
    entblk = jnp.sum(jnp.where(onehot & (bplus == wblk), ent, 0.0), axis=0,
                     keepdims=True)
    hit = wblk > 0
    wplus_ref[...] = jnp.where(hit, wblk, wplus_ref[...])
    entwin_ref[...] = jnp.where(hit, entblk, entwin_ref[...])

    @pl.when(i == _NSTEPS - 1)
    def _fin():
        do = (wplus_ref[...] > 0) & (entwin_ref[...] < worst_ref[...])
        do_ref[...] = do.astype(jnp.int32)
        cnt_ref[...] = jnp.sum(do.astype(jnp.int32), keepdims=True)


def _scatter_kernel(cnt_ref, wplus_ref, slot_ref, do_ref, mem_ref, feats_ref,
                    out_ref, buf, sem_in, sem_out):
    del mem_ref

    @pl.when(cnt_ref[0, 0] > 0)
    def _any():
        def body(c, carry):
            @pl.when(do_ref[0, c] > 0)
            def _write():
                b = wplus_ref[0, c] - 1
                s = slot_ref[0, c]
                cp = pltpu.make_async_copy(feats_ref.at[pl.ds(b, 1), :],
                                           buf, sem_in)
                cp.start()
                cp.wait()
                r = buf[...]
                buf[...] = r * lax.rsqrt(jnp.sum(r * r, keepdims=True))
                cp2 = pltpu.make_async_copy(buf,
                                            out_ref.at[c, pl.ds(s, 1), :],
                                            sem_out)
                cp2.start()
                cp2.wait()
            return carry
        lax.fori_loop(0, _C, body, 0)


@functools.partial(jax.jit, static_argnames=("interpret",))
def _impl(feature_memory, entropy_memory, logits, image_features_global,
          interpret=False):
    emt = entropy_memory.T
    wplus, slot, do, cnt = pl.pallas_call(
        _stats_kernel,
        grid=(_NSTEPS,),
        in_specs=[
            pl.BlockSpec((_BBLK, _C), lambda i: (i, 0)),
            pl.BlockSpec((_M, _C), lambda i: (0, 0)),
        ],
        out_specs=[
            pl.BlockSpec((1, _C), lambda i: (0, 0)),
            pl.BlockSpec((1, _C), lambda i: (0, 0)),
            pl.BlockSpec((1, _C), lambda i: (0, 0)),
            pl.BlockSpec((1, 1), lambda i: (0, 0)),
        ],
        out_shape=[
            jax.ShapeDtypeStruct((1, _C), jnp.int32),
            jax.ShapeDtypeStruct((1, _C), jnp.int32),
            jax.ShapeDtypeStruct((1, _C), jnp.int32),
            jax.ShapeDtypeStruct((1, 1), jnp.int32),
        ],
        scratch_shapes=[
            pltpu.VMEM((1, _C), jnp.float32),
            pltpu.VMEM((1, _C), jnp.float32),
        ],
        interpret=interpret,
    )(logits, emt)

    new_mem = pl.pallas_call(
        _scatter_kernel,
        in_specs=[
            pl.BlockSpec(memory_space=pltpu.SMEM),
            pl.BlockSpec(memory_space=pltpu.SMEM),
            pl.BlockSpec(memory_space=pltpu.SMEM),
            pl.BlockSpec(memory_space=pltpu.SMEM),
            pl.BlockSpec(memory_space=pltpu.MemorySpace.HBM),
            pl.BlockSpec(memory_space=pltpu.MemorySpace.HBM),
        ],
        out_specs=pl.BlockSpec(memory_space=pltpu.MemorySpace.HBM),
        out_shape=jax.ShapeDtypeStruct((_C, _M, _D), jnp.float32),
        scratch_shapes=[
            pltpu.VMEM((1, _D), jnp.float32),
            pltpu.SemaphoreType.DMA,
            pltpu.SemaphoreType.DMA,
        ],
        input_output_aliases={4: 0},
        interpret=interpret,
    )(cnt, wplus, slot, do, feature_memory, image_features_global)
    return new_mem


def kernel(feature_memory, entropy_memory, logits, image_features_global):
    return _impl(feature_memory, entropy_memory, logits,
                 image_features_global)
